# Initial kernel scaffold; baseline (speedup 1.0000x reference)
import jax
import jax.numpy as jnp
from jax.experimental import pallas as pl


def kernel(X, table):
    raise NotImplementedError("write your pallas kernel here")


# speedup vs baseline: 1.9861x; 1.9861x over previous
import functools

import numpy as np
import jax
import jax.numpy as jnp
from jax import lax
from jax.experimental import pallas as pl
from jax.experimental.pallas import tpu as pltpu
from jax.experimental.pallas import tpu_sc as plsc

B, T, F, VOCAB, D = 1024, 50, 26, 1000000, 16
NTOK = B * T
ROW = 1 + D
OUTF = ROW * F
NW = 32
TPW = NTOK // NW
G = 64
NCHUNK = TPW // G
IDS = G * F
NBATCH = IDS // 128
XROWS = G * 2 * F // 16
WROWS = XROWS + IDS
OROWS = G * OUTF // 16
EVR = IDS // 16


def _build_index_tables():
  p = np.arange(G * OUTF)
  g, r = p // OUTF, p % OUTF
  row, f = r // F, r % F
  src_cont = g * (2 * F) + f
  src_emb = XROWS * 16 + (g * F + f) * 16 + (row - 1)
  tfl = np.where(row == 0, src_cont, src_emb).astype(np.int32)
  q = np.arange(IDS)
  gq, fq = q // F, q % F
  efl = (gq * (2 * F) + F + fq).astype(np.int32)
  return tfl.reshape(OROWS, 16), efl.reshape(EVR, 16)


_TFL, _EFL = _build_index_tables()

@functools.cache
def _build_embed_sc():
  mesh = plsc.VectorSubcoreMesh(core_axis_name="c", subcore_axis_name="s")
  return functools.partial(
      pl.kernel,
      out_type=jax.ShapeDtypeStruct((NTOK * OUTF // 16, 16), jnp.float32),
      mesh=mesh,
      compiler_params=pltpu.CompilerParams(
          needs_layout_passes=False, use_tc_tiling_on_sc=False),
      scratch_types=[
          pltpu.VMEM((WROWS, 16), jnp.float32),
          pltpu.VMEM((OROWS, 16), jnp.float32),
          pltpu.VMEM((OROWS, 16), jnp.int32),
          pltpu.VMEM((EVR, 16), jnp.int32),
          pltpu.VMEM((NBATCH, 128), jnp.int32),
          pltpu.SemaphoreType.DMA,
      ],
  )(_embed_sc)


def _embed_sc(x_hbm, table_hbm, tfl_hbm, efl_hbm, out_hbm,
              work, outb, tfl, efl, idxb, sem):
  wid = lax.axis_index("s") * 2 + lax.axis_index("c")
  pltpu.sync_copy(tfl_hbm, tfl)
  pltpu.sync_copy(efl_hbm, efl)

  def chunk(c, carry):
    xrow0 = wid * (TPW * 2 * F // 16) + c * XROWS
    orow0 = wid * (TPW * OUTF // 16) + c * OROWS

    pltpu.sync_copy(x_hbm.at[pl.ds(xrow0, XROWS)], work.at[pl.ds(0, XROWS)])

    for k in range(EVR):
      ev = efl[k, :]
      v = plsc.load_gather(work, [ev >> 4, ev & 15])
      idxb[k // 8, pl.ds((k % 8) * 16, 16)] = v.astype(jnp.int32)

    handles = [
        pltpu.async_copy(
            table_hbm.at[idxb.at[j]],
            work.at[pl.ds(XROWS + j * 128, 128)],
            sem,
        )
        for j in range(NBATCH)
    ]
    for h in handles:
      h.wait()

    def tr(j, tcarry):
      sv = tfl[j, :]
      outb[j, :] = plsc.load_gather(work, [sv >> 4, sv & 15])
      return tcarry

    lax.fori_loop(0, OROWS, tr, 0)

    pltpu.sync_copy(outb, out_hbm.at[pl.ds(orow0, OROWS)])
    return carry

  lax.fori_loop(0, NCHUNK, chunk, 0)


def kernel(X, table):
  x2 = X.reshape(NTOK * 2 * F // 16, 16)
  out = _build_embed_sc()(x2, table, jnp.asarray(_TFL), jnp.asarray(_EFL))
  return out.reshape(B, T, ROW, F)


# speedup vs baseline: 2.2654x; 1.1406x over previous
import functools

import numpy as np
import jax
import jax.numpy as jnp
from jax import lax
from jax.experimental import pallas as pl
from jax.experimental.pallas import tpu as pltpu
from jax.experimental.pallas import tpu_sc as plsc

B, T, F, VOCAB, D = 1024, 50, 26, 1000000, 16
NTOK = B * T
ROW = 1 + D
OUTF = ROW * F
XF = 2 * F
NW = 32
TPW = NTOK // NW
G = 64
NCHUNK = TPW // G
IDS = G * F
NBATCH = IDS // 128
XLEN = G * XF
OLEN = G * OUTF
EVR = IDS // 16


def _build_index_tables():
  q = np.arange(IDS)
  gq, fq = q // F, q % F
  efl = (gq * XF + F + fq).astype(np.int32)
  csrc = (gq * XF + fq).astype(np.int32)
  cdst = (gq * OUTF + fq).astype(np.int32)
  return (efl.reshape(EVR, 16), csrc.reshape(EVR, 16), cdst.reshape(EVR, 16))


_EFL, _CSRC, _CDST = _build_index_tables()


@functools.cache
def _build_embed_sc():
  mesh = plsc.VectorSubcoreMesh(core_axis_name="c", subcore_axis_name="s")
  return functools.partial(
      pl.kernel,
      out_type=jax.ShapeDtypeStruct((NTOK * OUTF,), jnp.float32),
      mesh=mesh,
      compiler_params=pltpu.CompilerParams(
          needs_layout_passes=False, use_tc_tiling_on_sc=False),
      scratch_types=[
          pltpu.VMEM((XLEN,), jnp.float32),
          pltpu.VMEM((IDS, 16), jnp.float32),
          pltpu.VMEM((OLEN,), jnp.float32),
          pltpu.VMEM((NBATCH, 128), jnp.int32),
          pltpu.VMEM((EVR, 16), jnp.int32),
          pltpu.VMEM((EVR, 16), jnp.int32),
          pltpu.VMEM((EVR, 16), jnp.int32),
          pltpu.SemaphoreType.DMA,
      ],
  )(_embed_sc)


def _embed_sc(x_hbm, table_hbm, efl_hbm, csrc_hbm, cdst_hbm, out_hbm,
              xv, rows, outb, idxb, efl, csrc, cdst, sem):
  wid = lax.axis_index("s") * 2 + lax.axis_index("c")
  pltpu.sync_copy(efl_hbm, efl)
  pltpu.sync_copy(csrc_hbm, csrc)
  pltpu.sync_copy(cdst_hbm, cdst)

  def chunk(c, carry):
    xoff = wid * (TPW * XF) + c * XLEN
    ooff = wid * (TPW * OUTF) + c * OLEN

    pltpu.sync_copy(x_hbm.at[pl.ds(xoff, XLEN)], xv)

    for k in range(EVR):
      v = plsc.load_gather(xv, [efl[k, :]])
      idxb[k // 8, pl.ds((k % 8) * 16, 16)] = v.astype(jnp.int32)

    handles = [
        pltpu.async_copy(
            table_hbm.at[idxb.at[j]],
            rows.at[pl.ds(j * 128, 128)],
            sem,
        )
        for j in range(NBATCH)
    ]

    for k in range(EVR):
      v = plsc.load_gather(xv, [csrc[k, :]])
      plsc.store_scatter(outb, [cdst[k, :]], v)

    for h in handles:
      h.wait()

    dst0 = lax.iota(jnp.int32, 16) * F + F

    def gbody(g, dstv):
      base = g * F
      for f in range(F):
        plsc.store_scatter(outb, [dstv], rows[base + f, :])
        dstv = dstv + 1
      return dstv + (OUTF - F)

    lax.fori_loop(0, G, gbody, dst0)

    pltpu.sync_copy(outb, out_hbm.at[pl.ds(ooff, OLEN)])
    return carry

  lax.fori_loop(0, NCHUNK, chunk, 0)


def kernel(X, table):
  x1 = X.reshape(NTOK * XF)
  out = _build_embed_sc()(
      x1, table, jnp.asarray(_EFL), jnp.asarray(_CSRC), jnp.asarray(_CDST))
  return out.reshape(B, T, ROW, F)


# speedup vs baseline: 3.6281x; 1.6016x over previous
import functools

import jax
import jax.numpy as jnp
from jax import lax
from jax.experimental import pallas as pl
from jax.experimental.pallas import tpu as pltpu
from jax.experimental.pallas import tpu_sc as plsc

B, T, F, VOCAB, D = 1024, 50, 26, 1000000, 16
ROW = 1 + D
NBT = B // 128
NTT = (T + 7) // 8
TPAD = NTT * 8
NCHUNK = T * NBT
NW = 32
CPW = -(-NCHUNK // NW)
IDS = F * 128
EVR = IDS // 16


@functools.cache
def _build_embed_sc():
  mesh = plsc.VectorSubcoreMesh(core_axis_name="c", subcore_axis_name="s")
  return functools.partial(
      pl.kernel,
      out_type=jax.ShapeDtypeStruct((ROW, F, TPAD, B), jnp.float32),
      mesh=mesh,
      compiler_params=pltpu.CompilerParams(
          needs_layout_passes=False, use_tc_tiling_on_sc=False),
      scratch_types=[
          pltpu.VMEM((F, 256), jnp.float32),
          pltpu.VMEM((IDS, 16), jnp.float32),
          pltpu.VMEM((ROW, F, 128), jnp.float32),
          pltpu.VMEM((F, 128), jnp.int32),
          pltpu.SemaphoreType.DMA,
      ],
  )(_embed_sc)


def _embed_sc(x_hbm, tab_hbm, out_hbm, xbuf, rows, obuf, idxb, sem):
  w = lax.axis_index("s") * 2 + lax.axis_index("c")

  def chunk_body(i, carry):
    cid = i * NW + w

    @pl.when(cid < NCHUNK)
    def _():
      t = cid // NBT
      bt = cid % NBT

      pltpu.sync_copy(x_hbm.at[t, :, bt, :], xbuf)

      for k in range(EVR):
        f, j = k // 8, k % 8
        v = xbuf[f, pl.ds(128 + j * 16, 16)]
        idxb[f, pl.ds(j * 16, 16)] = v.astype(jnp.int32)

      handles = [
          pltpu.async_copy(
              tab_hbm.at[idxb.at[f]],
              rows.at[pl.ds(f * 128, 128)],
              sem,
          )
          for f in range(F)
      ]

      for k in range(EVR):
        f, j = k // 8, k % 8
        obuf[0, f, pl.ds(j * 16, 16)] = xbuf[f, pl.ds(j * 16, 16)]

      for h in handles:
        h.wait()

      rowv0 = lax.iota(jnp.int32, 16)

      def mbody(m, mc):
        rowv = rowv0 + m * 16
        f = m // 8
        col = (m % 8) * 16
        for r in range(1, ROW):
          v = plsc.load_gather(rows, [rowv, jnp.full((16,), r - 1, jnp.int32)])
          obuf[r, f, pl.ds(col, 16)] = v
        return mc

      lax.fori_loop(0, EVR, mbody, 0)

      pltpu.sync_copy(obuf, out_hbm.at[:, :, t, pl.ds(bt * 128, 128)])

    return carry

  lax.fori_loop(0, CPW, chunk_body, 0)


def kernel(X, table):
  x4 = (X.transpose(1, 3, 2, 0)
          .reshape(T, F, 2, NBT, 128)
          .transpose(0, 1, 3, 2, 4)
          .reshape(T, F, NBT, 256))
  out = _build_embed_sc()(x4, table)
  return out.transpose(3, 2, 0, 1)[:, :T]
